# Initial kernel scaffold; baseline (speedup 1.0000x reference)
import jax
import jax.numpy as jnp
from jax.experimental import pallas as pl


def kernel(input):
    raise NotImplementedError("write your pallas kernel here")


# speedup vs baseline: 5.4758x; 5.4758x over previous
import math

import jax
import jax.numpy as jnp
from jax.experimental import pallas as pl
from jax.experimental.pallas import tpu as pltpu

EMBEDDING_DIM = 1024
PADDING_IDX = 0

ROW_BLOCK = 512


def _sinusoid_kernel(inv_freq_ref, out_ref):
    i = pl.program_id(0)
    half = inv_freq_ref.shape[1]
    rows = out_ref.shape[1]
    pos = (
        jax.lax.broadcasted_iota(jnp.int32, (rows, 1), 0) + i * rows
    ).astype(jnp.float32)
    angle = pos * inv_freq_ref[0, :][None, :]
    s = jnp.sin(angle)
    c = jnp.cos(angle)
    tile = jnp.concatenate([s, c], axis=1)
    is_pad = (pos == float(PADDING_IDX))
    tile = jnp.where(is_pad, 0.0, tile)
    out_ref[...] = jnp.broadcast_to(tile[None], out_ref.shape)


def kernel(input):
    bsz, seq_len = input.shape
    half_dim = EMBEDDING_DIM // 2
    scale = math.log(10000.0) / (half_dim - 1)
    inv_freq = jnp.exp(
        jnp.arange(half_dim, dtype=jnp.float32) * -scale
    ).reshape(1, half_dim)

    n_blocks = seq_len // ROW_BLOCK
    out = pl.pallas_call(
        _sinusoid_kernel,
        grid=(n_blocks,),
        in_specs=[
            pl.BlockSpec((1, half_dim), lambda i: (0, 0)),
        ],
        out_specs=pl.BlockSpec(
            (bsz, ROW_BLOCK, EMBEDDING_DIM), lambda i: (0, i, 0)
        ),
        out_shape=jax.ShapeDtypeStruct(
            (bsz, seq_len, EMBEDDING_DIM), jnp.float32
        ),
    )(inv_freq)
    return out


# speedup vs baseline: 7.6766x; 1.4019x over previous
import math

import jax
import jax.numpy as jnp
from jax.experimental import pallas as pl
from jax.experimental.pallas import tpu as pltpu

EMBEDDING_DIM = 1024
PADDING_IDX = 0

ROW_BLOCK = 512


def _sinusoid_kernel(inv_freq_ref, out_ref, base_ref):
    i = pl.program_id(0)
    half = inv_freq_ref.shape[1]
    rows = out_ref.shape[1]
    w = inv_freq_ref[0, :][None, :]

    @pl.when(i == 0)
    def _init_base():
        dp = jax.lax.broadcasted_iota(jnp.int32, (rows, 1), 0).astype(
            jnp.float32
        )
        d = dp * w
        base_ref[:, :half] = jnp.sin(d)
        base_ref[:, half:] = jnp.cos(d)

    sd = base_ref[:, :half]
    cd = base_ref[:, half:]
    b = (i * rows).astype(jnp.float32) * w
    sb = jnp.sin(b)
    cb = jnp.cos(b)
    tile_sin = sd * cb + cd * sb
    tile_cos = cd * cb - sd * sb
    tile = jnp.concatenate([tile_sin, tile_cos], axis=1)
    out_ref[...] = jnp.broadcast_to(tile[None], out_ref.shape)

    @pl.when(i == 0)
    def _zero_pad_row():
        out_ref[:, PADDING_IDX : PADDING_IDX + 1, :] = jnp.zeros(
            (out_ref.shape[0], 1, out_ref.shape[2]), jnp.float32
        )


def kernel(input):
    bsz, seq_len = input.shape
    half_dim = EMBEDDING_DIM // 2
    scale = math.log(10000.0) / (half_dim - 1)
    inv_freq = jnp.exp(
        jnp.arange(half_dim, dtype=jnp.float32) * -scale
    ).reshape(1, half_dim)

    n_blocks = seq_len // ROW_BLOCK
    out = pl.pallas_call(
        _sinusoid_kernel,
        grid=(n_blocks,),
        in_specs=[
            pl.BlockSpec((1, half_dim), lambda i: (0, 0)),
        ],
        out_specs=pl.BlockSpec(
            (bsz, ROW_BLOCK, EMBEDDING_DIM), lambda i: (0, i, 0)
        ),
        out_shape=jax.ShapeDtypeStruct(
            (bsz, seq_len, EMBEDDING_DIM), jnp.float32
        ),
        scratch_shapes=[
            pltpu.VMEM((ROW_BLOCK, EMBEDDING_DIM), jnp.float32)
        ],
    )(inv_freq)
    return out


# speedup vs baseline: 8.1864x; 1.0664x over previous
import math

import jax
import jax.numpy as jnp
from jax.experimental import pallas as pl
from jax.experimental.pallas import tpu as pltpu

EMBEDDING_DIM = 1024
PADDING_IDX = 0

ROW_BLOCK = 256


def _sinusoid_kernel(inv_freq_ref, out_ref, base_ref):
    i = pl.program_id(0)
    half = inv_freq_ref.shape[1]
    rows = out_ref.shape[1]
    w = inv_freq_ref[0, :][None, :]

    @pl.when(i == 0)
    def _init_base():
        dp = jax.lax.broadcasted_iota(jnp.int32, (rows, 1), 0).astype(
            jnp.float32
        )
        d = dp * w
        base_ref[:, :half] = jnp.sin(d)
        base_ref[:, half:] = jnp.cos(d)

    sd = base_ref[:, :half]
    cd = base_ref[:, half:]
    b = (i * rows).astype(jnp.float32) * w
    sb = jnp.sin(b)
    cb = jnp.cos(b)
    tile_sin = sd * cb + cd * sb
    tile_cos = cd * cb - sd * sb
    tile = jnp.concatenate([tile_sin, tile_cos], axis=1)
    out_ref[...] = jnp.broadcast_to(tile[None], out_ref.shape)

    @pl.when(i == 0)
    def _zero_pad_row():
        out_ref[:, PADDING_IDX : PADDING_IDX + 1, :] = jnp.zeros(
            (out_ref.shape[0], 1, out_ref.shape[2]), jnp.float32
        )


def kernel(input):
    bsz, seq_len = input.shape
    half_dim = EMBEDDING_DIM // 2
    scale = math.log(10000.0) / (half_dim - 1)
    inv_freq = jnp.exp(
        jnp.arange(half_dim, dtype=jnp.float32) * -scale
    ).reshape(1, half_dim)

    n_blocks = seq_len // ROW_BLOCK
    out = pl.pallas_call(
        _sinusoid_kernel,
        grid=(n_blocks,),
        in_specs=[
            pl.BlockSpec((1, half_dim), lambda i: (0, 0)),
        ],
        out_specs=pl.BlockSpec(
            (bsz, ROW_BLOCK, EMBEDDING_DIM), lambda i: (0, i, 0)
        ),
        out_shape=jax.ShapeDtypeStruct(
            (bsz, seq_len, EMBEDDING_DIM), jnp.float32
        ),
        scratch_shapes=[
            pltpu.VMEM((ROW_BLOCK, EMBEDDING_DIM), jnp.float32)
        ],
    )(inv_freq)
    return out
